# Initial kernel scaffold; baseline (speedup 1.0000x reference)
import jax
import jax.numpy as jnp
from jax.experimental import pallas as pl


def kernel(xyz1, xyz2, points1, points2, W1, b1, g1, be1, W2, b2, g2, be2):
    raise NotImplementedError("write your pallas kernel here")


# speedup vs baseline: 31.1777x; 31.1777x over previous
import functools

import jax
import jax.numpy as jnp
from jax.experimental import pallas as pl

B, N1, N2, C1, C2 = 4, 4096, 1024, 128, 256
H1, H2 = 256, 128
IN_CH = C1 + C2
BQ = 512
BQ2 = 2048
NTOT = B * N1
EPS_D = 1e-10
EPS_BN = 1e-5


def _stage1_kernel(xyz1t_ref, xyz2t_ref, p1_ref, p2_ref, W1_ref, b1_ref,
                   y1_ref, ssum_ref, ssq_ref):
    b = pl.program_id(0)
    i = pl.program_id(1)

    @pl.when(jnp.logical_and(b == 0, i == 0))
    def _init():
        ssum_ref[...] = jnp.zeros_like(ssum_ref)
        ssq_ref[...] = jnp.zeros_like(ssq_ref)

    q = xyz1t_ref[0]
    p = xyz2t_ref[0]
    d = (p[0][:, None] - q[0][None, :]) ** 2
    d = d + (p[1][:, None] - q[1][None, :]) ** 2
    d = d + (p[2][:, None] - q[2][None, :]) ** 2
    d = jnp.maximum(d, EPS_D)
    m1 = jnp.min(d, axis=0)
    d1 = jnp.where(d <= m1[None, :], jnp.inf, d)
    m2 = jnp.min(d1, axis=0)
    d2 = jnp.where(d1 <= m2[None, :], jnp.inf, d1)
    m3 = jnp.min(d2, axis=0)
    w = jnp.where(d <= m3[None, :], 1.0 / d, 0.0)
    wsum = jnp.sum(w, axis=0)
    interp = jnp.dot(p2_ref[0], w, preferred_element_type=jnp.float32)
    interp = interp * (1.0 / wsum)[None, :]
    x = jnp.concatenate([p1_ref[0], interp], axis=0)
    y1 = jnp.dot(W1_ref[...], x, preferred_element_type=jnp.float32)
    y1 = y1 + b1_ref[...]
    y1_ref[0] = y1
    ssum_ref[...] += jnp.sum(y1, axis=1, keepdims=True)
    ssq_ref[...] += jnp.sum(y1 * y1, axis=1, keepdims=True)


def _stage2_kernel(y1_ref, W2_ref, g1_ref, be1_ref, b2_ref, s_ref, sq_ref,
                   y2_ref, ssum_ref, ssq_ref):
    b = pl.program_id(0)
    i = pl.program_id(1)

    @pl.when(jnp.logical_and(b == 0, i == 0))
    def _init():
        ssum_ref[...] = jnp.zeros_like(ssum_ref)
        ssq_ref[...] = jnp.zeros_like(ssq_ref)

    mean = s_ref[...] * (1.0 / NTOT)
    var = sq_ref[...] * (1.0 / NTOT) - mean * mean
    scale = g1_ref[...] / jnp.sqrt(var + EPS_BN)
    h = (y1_ref[0] - mean) * scale + be1_ref[...]
    h = jnp.maximum(h, 0.0)
    y2 = jnp.dot(W2_ref[...], h, preferred_element_type=jnp.float32)
    y2 = y2 + b2_ref[...]
    y2_ref[0] = y2
    ssum_ref[...] += jnp.sum(y2, axis=1, keepdims=True)
    ssq_ref[...] += jnp.sum(y2 * y2, axis=1, keepdims=True)


def _stage3_kernel(y2_ref, g2_ref, be2_ref, s_ref, sq_ref, out_ref):
    mean = s_ref[...] * (1.0 / NTOT)
    var = sq_ref[...] * (1.0 / NTOT) - mean * mean
    scale = g2_ref[...] / jnp.sqrt(var + EPS_BN)
    o = (y2_ref[0] - mean) * scale + be2_ref[...]
    out_ref[0] = jnp.maximum(o, 0.0)


def _fp_impl(xyz1, xyz2, points1, points2, W1, b1, g1, be1, W2, b2, g2, be2,
             interpret=False):
    xyz1t = jnp.transpose(xyz1, (0, 2, 1))
    xyz2t = jnp.transpose(xyz2, (0, 2, 1))
    b1c = b1[:, None]
    g1c = g1[:, None]
    be1c = be1[:, None]
    b2c = b2[:, None]
    g2c = g2[:, None]
    be2c = be2[:, None]

    f32 = jnp.float32
    y1, s1, q1 = pl.pallas_call(
        _stage1_kernel,
        grid=(B, N1 // BQ),
        in_specs=[
            pl.BlockSpec((1, 3, BQ), lambda b, i: (b, 0, i)),
            pl.BlockSpec((1, 3, N2), lambda b, i: (b, 0, 0)),
            pl.BlockSpec((1, C1, BQ), lambda b, i: (b, 0, i)),
            pl.BlockSpec((1, C2, N2), lambda b, i: (b, 0, 0)),
            pl.BlockSpec((H1, IN_CH), lambda b, i: (0, 0)),
            pl.BlockSpec((H1, 1), lambda b, i: (0, 0)),
        ],
        out_specs=[
            pl.BlockSpec((1, H1, BQ), lambda b, i: (b, 0, i)),
            pl.BlockSpec((H1, 1), lambda b, i: (0, 0)),
            pl.BlockSpec((H1, 1), lambda b, i: (0, 0)),
        ],
        out_shape=[
            jax.ShapeDtypeStruct((B, H1, N1), f32),
            jax.ShapeDtypeStruct((H1, 1), f32),
            jax.ShapeDtypeStruct((H1, 1), f32),
        ],
        interpret=interpret,
    )(xyz1t, xyz2t, points1, points2, W1, b1c)

    y2, s2, q2 = pl.pallas_call(
        _stage2_kernel,
        grid=(B, N1 // BQ2),
        in_specs=[
            pl.BlockSpec((1, H1, BQ2), lambda b, i: (b, 0, i)),
            pl.BlockSpec((H2, H1), lambda b, i: (0, 0)),
            pl.BlockSpec((H1, 1), lambda b, i: (0, 0)),
            pl.BlockSpec((H1, 1), lambda b, i: (0, 0)),
            pl.BlockSpec((H2, 1), lambda b, i: (0, 0)),
            pl.BlockSpec((H1, 1), lambda b, i: (0, 0)),
            pl.BlockSpec((H1, 1), lambda b, i: (0, 0)),
        ],
        out_specs=[
            pl.BlockSpec((1, H2, BQ2), lambda b, i: (b, 0, i)),
            pl.BlockSpec((H2, 1), lambda b, i: (0, 0)),
            pl.BlockSpec((H2, 1), lambda b, i: (0, 0)),
        ],
        out_shape=[
            jax.ShapeDtypeStruct((B, H2, N1), f32),
            jax.ShapeDtypeStruct((H2, 1), f32),
            jax.ShapeDtypeStruct((H2, 1), f32),
        ],
        interpret=interpret,
    )(y1, W2, g1c, be1c, b2c, s1, q1)

    out = pl.pallas_call(
        _stage3_kernel,
        grid=(B, N1 // BQ2),
        in_specs=[
            pl.BlockSpec((1, H2, BQ2), lambda b, i: (b, 0, i)),
            pl.BlockSpec((H2, 1), lambda b, i: (0, 0)),
            pl.BlockSpec((H2, 1), lambda b, i: (0, 0)),
            pl.BlockSpec((H2, 1), lambda b, i: (0, 0)),
            pl.BlockSpec((H2, 1), lambda b, i: (0, 0)),
        ],
        out_specs=pl.BlockSpec((1, H2, BQ2), lambda b, i: (b, 0, i)),
        out_shape=jax.ShapeDtypeStruct((B, H2, N1), f32),
        interpret=interpret,
    )(y2, g2c, be2c, s2, q2)
    return out


def kernel(xyz1, xyz2, points1, points2, W1, b1, g1, be1, W2, b2, g2, be2):
    return _fp_impl(xyz1, xyz2, points1, points2, W1, b1, g1, be1,
                    W2, b2, g2, be2)
